# baseline (speedup 1.0000x reference)
import functools

import jax
import jax.numpy as jnp
from jax import lax
from jax.experimental import pallas as pl
from jax.experimental.pallas import tpu as pltpu
from jax.experimental.pallas import tpu_sc as plsc

B = 16384
D = 64
NC = 2
NS = 16
NW = NC * NS
BPW = B // NW
CHUNK = 128
NCHUNK = BPW // CHUNK


NB = 8


def _gather_body(idx_hbm, tab_hbm, out_hbm, idx_v, blocks_v, rows16_v,
                 sem_flush, *sems):
    wid = lax.axis_index("s") * NC + lax.axis_index("c")
    base = wid * BPW
    pltpu.sync_copy(idx_hbm.at[wid], idx_v)
    rowq = lax.iota(jnp.int32, 16)

    def fire(iv, lane, slot):
        cb = pl.multiple_of((iv[lane] >> 7) * 128, 128)
        pltpu.make_async_copy(tab_hbm.at[:, pl.ds(cb, 128)],
                              blocks_v.at[slot], sems[slot]).start()

    def extract(iv, lane, slot, p):
        colv = jnp.full((16,), iv[lane] & 127, jnp.int32)
        for q in range(4):
            vals = plsc.load_gather(blocks_v.at[slot], [rowq + 16 * q, colv])
            rows16_v[p, lane, pl.ds(16 * q, 16)] = vals

    def wait(slot):
        pltpu.make_async_copy(tab_hbm.at[:, pl.ds(0, 128)],
                              blocks_v.at[slot], sems[slot]).wait()

    nr = BPW // 16

    def wait_flush(p):
        pltpu.make_async_copy(out_hbm.at[pl.ds(0, 16)], rows16_v.at[p],
                              sem_flush).wait()

    def superround(r, p):
        iv = idx_v[pl.ds(r * 16, 16)]
        ivn = idx_v[pl.ds(jnp.minimum(r + 1, nr - 1) * 16, 16)]
        for b in range(NB):
            wait(b)
            extract(iv, b, b, p)
            fire(iv, NB + b, b)
        for b in range(NB):
            wait(b)
            extract(iv, NB + b, b, p)

            @pl.when(r < nr - 1)
            def _refill():
                fire(ivn, b, b)

        pltpu.make_async_copy(rows16_v.at[p],
                              out_hbm.at[pl.ds(base + r * 16, 16)],
                              sem_flush).start()

    def dround(g, _):
        for p in range(2):
            @pl.when(g > 0)
            def _drain():
                wait_flush(p)

            superround(2 * g + p, p)
        return _

    iv0 = idx_v[pl.ds(0, 16)]
    for b in range(NB):
        fire(iv0, b, b)
    lax.fori_loop(0, nr // 2, dround, 0)
    wait_flush(0)
    wait_flush(1)


@functools.lru_cache(maxsize=1)
def _sc_gather():
    return pl.kernel(
        _gather_body,
        out_type=pltpu.HBM((B, D), jnp.float32),
        mesh=plsc.VectorSubcoreMesh(core_axis_name="c", subcore_axis_name="s",
                                    num_cores=NC, num_subcores=NS),
        scratch_types=[
            pltpu.VMEM((BPW,), jnp.int32),
            pltpu.VMEM((NB, D, 128), jnp.float32),
            pltpu.VMEM((2, 16, D), jnp.float32),
            pltpu.SemaphoreType.DMA,
        ] + [pltpu.SemaphoreType.DMA] * NB,
        compiler_params=pltpu.CompilerParams(needs_layout_passes=False),
    )


def _bn_relu(h, g, be, eps=1e-5):
    mean = jnp.mean(h, axis=0, keepdims=True)
    c = h - mean
    var = jnp.mean(c * c, axis=0, keepdims=True)
    return jnp.maximum(c * lax.rsqrt(var + eps) * g + be, 0.0)


def _mlp_body(u_ref, v_ref, W1_ref, b1_ref, g1_ref, be1_ref,
              W2_ref, b2_ref, g2_ref, be2_ref,
              W3_ref, b3_ref, g3_ref, be3_ref,
              W4_ref, b4_ref, out_ref):
    x = u_ref[...] * v_ref[...]
    h = jnp.dot(x, W1_ref[...], preferred_element_type=jnp.float32) + b1_ref[...]
    h = _bn_relu(h, g1_ref[...], be1_ref[...])
    h = jnp.dot(h, W2_ref[...], preferred_element_type=jnp.float32) + b2_ref[...]
    h = _bn_relu(h, g2_ref[...], be2_ref[...])
    h = jnp.dot(h, W3_ref[...], preferred_element_type=jnp.float32) + b3_ref[...]
    h = _bn_relu(h, g3_ref[...], be3_ref[...])
    out_ref[...] = (jnp.dot(h, W4_ref[...], preferred_element_type=jnp.float32)
                    + b4_ref[...])


_tc_mlp = pl.pallas_call(
    _mlp_body,
    out_shape=jax.ShapeDtypeStruct((B, 1), jnp.float32),
)


def kernel(user, video, user_table, video_table,
           W1, b1, g1, be1, W2, b2, g2, be2, W3, b3, g3, be3, W4, b4):
    u_idx = user.astype(jnp.int32).reshape(NW, BPW)
    v_idx = video.astype(jnp.int32).reshape(NW, BPW)
    gather = _sc_gather()
    u_emb = gather(u_idx, jnp.transpose(user_table))
    v_emb = gather(v_idx, jnp.transpose(video_table))
    out = _tc_mlp(u_emb, v_emb, W1, b1, g1, be1, W2, b2, g2, be2,
                  W3, b3, g3, be3, W4, b4)
    return jnp.squeeze(out, axis=-1)


# speedup vs baseline: 1.0257x; 1.0257x over previous
import functools

import jax
import jax.numpy as jnp
from jax import lax
from jax.experimental import pallas as pl
from jax.experimental.pallas import tpu as pltpu
from jax.experimental.pallas import tpu_sc as plsc

B = 16384
D = 64
NC = 2
NS = 16
NW = NC * NS
BPW = B // NW
CHUNK = 128
NCHUNK = BPW // CHUNK


NB = 8


def _gather_body(idx_hbm, utab_hbm, vtab_hbm, uout_hbm, vout_hbm,
                 idx_v, blocks_v, rows16_v, sem_flush, *sems):
    wid = lax.axis_index("s") * NC + lax.axis_index("c")
    base = wid * BPW
    pltpu.sync_copy(idx_hbm.at[0, wid], idx_v.at[pl.ds(0, BPW)])
    pltpu.sync_copy(idx_hbm.at[1, wid], idx_v.at[pl.ds(BPW, BPW)])
    rowq = lax.iota(jnp.int32, 16)
    nru = BPW // 16

    def fire_t(tab_hbm, iv, lane, slot):
        cb = pl.multiple_of((iv[lane] >> 7) * 128, 128)
        pltpu.make_async_copy(tab_hbm.at[:, pl.ds(cb, 128)],
                              blocks_v.at[slot], sems[slot]).start()

    def fire(r, iv, lane, slot):
        @pl.when(r < nru)
        def _u():
            fire_t(utab_hbm, iv, lane, slot)

        @pl.when(r >= nru)
        def _v():
            fire_t(vtab_hbm, iv, lane, slot)

    def extract(iv, lane, slot, p):
        colv = jnp.full((16,), iv[lane] & 127, jnp.int32)
        for q in range(4):
            vals = plsc.load_gather(blocks_v.at[slot], [rowq + 16 * q, colv])
            rows16_v[p, lane, pl.ds(16 * q, 16)] = vals

    def wait(slot):
        pltpu.make_async_copy(utab_hbm.at[:, pl.ds(0, 128)],
                              blocks_v.at[slot], sems[slot]).wait()

    nr = 2 * BPW // 16

    def wait_flush(p):
        pltpu.make_async_copy(uout_hbm.at[pl.ds(0, 16)], rows16_v.at[p],
                              sem_flush).wait()

    def superround(r, p):
        iv = idx_v[pl.ds(r * 16, 16)]
        rn = jnp.minimum(r + 1, nr - 1)
        ivn = idx_v[pl.ds(rn * 16, 16)]
        for b in range(NB):
            wait(b)
            extract(iv, b, b, p)
            fire(r, iv, NB + b, b)
        for b in range(NB):
            wait(b)
            extract(iv, NB + b, b, p)

            @pl.when(r < nr - 1)
            def _refill():
                fire(rn, ivn, b, b)

        @pl.when(r < nru)
        def _fu():
            pltpu.make_async_copy(rows16_v.at[p],
                                  uout_hbm.at[pl.ds(base + r * 16, 16)],
                                  sem_flush).start()

        @pl.when(r >= nru)
        def _fv():
            pltpu.make_async_copy(
                rows16_v.at[p],
                vout_hbm.at[pl.ds(base + (r - nru) * 16, 16)],
                sem_flush).start()

    def dround(g, _):
        for p in range(2):
            @pl.when(g > 0)
            def _drain():
                wait_flush(p)

            superround(2 * g + p, p)
        return _

    iv0 = idx_v[pl.ds(0, 16)]
    for b in range(NB):
        fire_t(utab_hbm, iv0, b, b)
    lax.fori_loop(0, nr // 2, dround, 0)
    wait_flush(0)
    wait_flush(1)


@functools.lru_cache(maxsize=1)
def _sc_gather():
    return pl.kernel(
        _gather_body,
        out_type=(pltpu.HBM((B, D), jnp.float32),
                  pltpu.HBM((B, D), jnp.float32)),
        mesh=plsc.VectorSubcoreMesh(core_axis_name="c", subcore_axis_name="s",
                                    num_cores=NC, num_subcores=NS),
        scratch_types=[
            pltpu.VMEM((2 * BPW,), jnp.int32),
            pltpu.VMEM((NB, D, 128), jnp.float32),
            pltpu.VMEM((2, 16, D), jnp.float32),
            pltpu.SemaphoreType.DMA,
        ] + [pltpu.SemaphoreType.DMA] * NB,
        compiler_params=pltpu.CompilerParams(needs_layout_passes=False),
    )


def _bn_relu(h, g, be, eps=1e-5):
    mean = jnp.mean(h, axis=0, keepdims=True)
    c = h - mean
    var = jnp.mean(c * c, axis=0, keepdims=True)
    return jnp.maximum(c * lax.rsqrt(var + eps) * g + be, 0.0)


def _mlp_body(u_ref, v_ref, W1_ref, b1_ref, g1_ref, be1_ref,
              W2_ref, b2_ref, g2_ref, be2_ref,
              W3_ref, b3_ref, g3_ref, be3_ref,
              W4_ref, b4_ref, out_ref):
    x = u_ref[...] * v_ref[...]
    h = jnp.dot(x, W1_ref[...], preferred_element_type=jnp.float32) + b1_ref[...]
    h = _bn_relu(h, g1_ref[...], be1_ref[...])
    h = jnp.dot(h, W2_ref[...], preferred_element_type=jnp.float32) + b2_ref[...]
    h = _bn_relu(h, g2_ref[...], be2_ref[...])
    h = jnp.dot(h, W3_ref[...], preferred_element_type=jnp.float32) + b3_ref[...]
    h = _bn_relu(h, g3_ref[...], be3_ref[...])
    out_ref[...] = (jnp.dot(h, W4_ref[...], preferred_element_type=jnp.float32)
                    + b4_ref[...])


_tc_mlp = pl.pallas_call(
    _mlp_body,
    out_shape=jax.ShapeDtypeStruct((B, 1), jnp.float32),
)


def kernel(user, video, user_table, video_table,
           W1, b1, g1, be1, W2, b2, g2, be2, W3, b3, g3, be3, W4, b4):
    uv_idx = jnp.stack([user.astype(jnp.int32).reshape(NW, BPW),
                        video.astype(jnp.int32).reshape(NW, BPW)])
    u_emb, v_emb = _sc_gather()(uv_idx, jnp.transpose(user_table),
                                jnp.transpose(video_table))
    out = _tc_mlp(u_emb, v_emb, W1, b1, g1, be1, W2, b2, g2, be2,
                  W3, b3, g3, be3, W4, b4)
    return jnp.squeeze(out, axis=-1)
